# Initial kernel scaffold; baseline (speedup 1.0000x reference)
import jax
import jax.numpy as jnp
from jax.experimental import pallas as pl


def kernel(logits):
    raise NotImplementedError("write your pallas kernel here")


# speedup vs baseline: 1.7562x; 1.7562x over previous
import functools

import jax
import jax.numpy as jnp
import numpy as np
from jax import lax
from jax.experimental import pallas as pl
from jax.experimental.pallas import tpu as pltpu
from jax.experimental.pallas import tpu_sc as plsc

_TAU = 0.5
_K = 10
_EPS = float(np.finfo(np.float32).eps)

_B, _N = 64, 32768
_L = 16
_NSL = _N // _L
_NC, _NS = 2, 16
_NW = _NC * _NS
_RPW = _B // _NW
_PREP_ROWS = 8


def _prep_body(x_ref, u_ref, w_ref):
    z = x_ref[...] - jnp.log(-jnp.log(u_ref[...]))
    m = jnp.max(z, axis=-1, keepdims=True)
    w = jnp.exp((z - m) * (1.0 / _TAU))
    w_ref[...] = w / jnp.sum(w, axis=-1, keepdims=True)


def _prep(x, u):
    return pl.pallas_call(
        _prep_body,
        grid=(_B // _PREP_ROWS,),
        in_specs=[
            pl.BlockSpec((_PREP_ROWS, _N), lambda i: (i, 0)),
            pl.BlockSpec((_PREP_ROWS, _N), lambda i: (i, 0)),
        ],
        out_specs=pl.BlockSpec((_PREP_ROWS, _N), lambda i: (i, 0)),
        out_shape=jax.ShapeDtypeStruct((_B, _N), jnp.float32),
    )(x, u)


def _sc_iterate(w):
    mesh = plsc.VectorSubcoreMesh(core_axis_name="c", subcore_axis_name="s")

    @functools.partial(
        pl.kernel,
        mesh=mesh,
        out_type=jax.ShapeDtypeStruct((_B, _N), jnp.float32),
        compiler_params=pltpu.CompilerParams(needs_layout_passes=False),
        scratch_types=[
            pltpu.VMEM((_N,), jnp.float32),
            pltpu.VMEM((_N,), jnp.float32),
            pltpu.VMEM((_L,), jnp.float32),
        ],
    )
    def run(w_hbm, out_hbm, wbuf, kbuf, sbuf):
        wid = lax.axis_index("s") * _NC + lax.axis_index("c")
        zeros = jnp.zeros((_L,), jnp.float32)
        lanes = lax.iota(jnp.int32, _L)

        def lane_sum(v):
            for sh in (8, 4, 2, 1):
                sbuf[...] = v
                v = v + plsc.load_gather(sbuf, [lanes ^ sh])
            return v

        for j in range(_RPW):
            row = wid * _RPW + j
            pltpu.sync_copy(w_hbm.at[row], wbuf)

            def first_body(i, vsum):
                sl = pl.ds(i * _L, _L)
                p = wbuf[sl]
                kbuf[sl] = p
                mask = jnp.maximum(1.0 - p, _EPS)
                w2 = p * (mask * mask)
                wbuf[sl] = w2
                return vsum + w2

            vsum = lax.fori_loop(0, _NSL, first_body, zeros, unroll=8)
            s = lane_sum(vsum)

            def round_body(t, s):
                inv = 1.0 / s

                def body(i, vsum):
                    sl = pl.ds(i * _L, _L)
                    p = wbuf[sl] * inv
                    kbuf[sl] = kbuf[sl] + p
                    mask = jnp.maximum(1.0 - p, _EPS)
                    w2 = p * (mask * mask)
                    wbuf[sl] = w2
                    return vsum + w2

                vsum = lax.fori_loop(0, _NSL, body, zeros, unroll=8)
                return lane_sum(vsum)

            lax.fori_loop(0, _K - 1, round_body, s)
            pltpu.sync_copy(kbuf, out_hbm.at[row])

    return run(w)


def kernel(logits):
    x = jnp.squeeze(logits, 2)
    key = jax.random.fold_in(jax.random.key(0), 7)
    u = jax.random.uniform(key, x.shape, dtype=jnp.float32,
                           minval=_EPS, maxval=1.0)
    w = _prep(x, u)
    khot = _sc_iterate(w)
    return jnp.expand_dims(khot, -1)


# speedup vs baseline: 4.3481x; 2.4758x over previous
import functools

import jax
import jax.numpy as jnp
import numpy as np
from jax import lax
from jax.experimental import pallas as pl
from jax.experimental.pallas import tpu as pltpu
from jax.experimental.pallas import tpu_sc as plsc

_TAU = 0.5
_K = 10
_EPS = float(np.finfo(np.float32).eps)

_B, _N = 64, 32768
_L = 16
_NSL = _N // _L
_NC, _NS = 2, 16
_NW = _NC * _NS
_RPW = _B // _NW
_PREP_ROWS = 8


def _prep_body(x_ref, u_ref, w_ref):
    z = x_ref[...] - jnp.log(-jnp.log(u_ref[...]))
    m = jnp.max(z, axis=-1, keepdims=True)
    w = jnp.exp((z - m) * (1.0 / _TAU))
    w_ref[...] = w / jnp.sum(w, axis=-1, keepdims=True)


def _prep(x, u):
    return pl.pallas_call(
        _prep_body,
        grid=(_B // _PREP_ROWS,),
        in_specs=[
            pl.BlockSpec((_PREP_ROWS, _N), lambda i: (i, 0)),
            pl.BlockSpec((_PREP_ROWS, _N), lambda i: (i, 0)),
        ],
        out_specs=pl.BlockSpec((_PREP_ROWS, _N), lambda i: (i, 0)),
        out_shape=jax.ShapeDtypeStruct((_B, _N), jnp.float32),
    )(x, u)


def _sc_iterate(w):
    mesh = plsc.VectorSubcoreMesh(core_axis_name="c", subcore_axis_name="s")

    @functools.partial(
        pl.kernel,
        mesh=mesh,
        out_type=jax.ShapeDtypeStruct((_B, _N), jnp.float32),
        compiler_params=pltpu.CompilerParams(needs_layout_passes=False),
        scratch_types=[
            pltpu.VMEM((_N,), jnp.float32),
            pltpu.VMEM((_N,), jnp.float32),
            pltpu.VMEM((_L,), jnp.float32),
        ],
    )
    def run(w_hbm, out_hbm, wbuf, kbuf, sbuf):
        wid = lax.axis_index("s") * _NC + lax.axis_index("c")
        zeros = jnp.zeros((_L,), jnp.float32)
        lanes = lax.iota(jnp.int32, _L)

        def lane_sum(v):
            for sh in (8, 4, 2, 1):
                sbuf[...] = v
                v = v + plsc.load_gather(sbuf, [lanes ^ sh])
            return v

        for j in range(_RPW):
            row = wid * _RPW + j
            pltpu.sync_copy(w_hbm.at[row], wbuf)

            def first_body(off, vsum):
                sl = pl.ds(off, _L)
                p = wbuf[sl]
                kbuf[sl] = p
                mask = jnp.maximum(1.0 - p, _EPS)
                w2 = p * (mask * mask)
                wbuf[sl] = w2
                return vsum + w2

            vsum = plsc.parallel_loop(0, _N, _L, unroll=8,
                                      carry=zeros)(first_body)
            s = lane_sum(vsum)

            def round_body(t, s):
                inv = 1.0 / s

                def body(off, vsum):
                    sl = pl.ds(off, _L)
                    p = wbuf[sl] * inv
                    kbuf[sl] = kbuf[sl] + p
                    mask = jnp.maximum(1.0 - p, _EPS)
                    w2 = p * (mask * mask)
                    wbuf[sl] = w2
                    return vsum + w2

                vsum = plsc.parallel_loop(0, _N, _L, unroll=8,
                                          carry=zeros)(body)
                return lane_sum(vsum)

            lax.fori_loop(0, _K - 1, round_body, s)
            pltpu.sync_copy(kbuf, out_hbm.at[row])

    return run(w)


def kernel(logits):
    x = jnp.squeeze(logits, 2)
    key = jax.random.fold_in(jax.random.key(0), 7)
    u = jax.random.uniform(key, x.shape, dtype=jnp.float32,
                           minval=_EPS, maxval=1.0)
    w = _prep(x, u)
    khot = _sc_iterate(w)
    return jnp.expand_dims(khot, -1)
